# baseline (speedup 1.0000x reference)
import functools

import jax
import jax.numpy as jnp
from jax import lax
from jax.experimental import pallas as pl
from jax.experimental.pallas import tpu as pltpu
from jax.experimental.pallas import tpu_sc as plsc

NF = 26
NV = 100001
EMB = 32
BATCH = 16384
DW = NF * EMB
R = BATCH * NF
NW = 32
RPW = R // NW
GROUP = 128
NBUF_G = 8
BUFROWS = GROUP * NBUF_G
NOUT = RPW // BUFROWS
BB = 1024
NBLK = BATCH // BB
EPS = 1e-5


def _sc_gather(table_flat, idx3):
    mesh = plsc.VectorSubcoreMesh(core_axis_name="c", subcore_axis_name="s")

    @functools.partial(
        pl.kernel,
        out_type=jax.ShapeDtypeStruct((R, EMB), jnp.float32),
        mesh=mesh,
        compiler_params=pltpu.CompilerParams(use_tc_tiling_on_sc=False),
        scratch_types=[
            pltpu.VMEM((RPW // GROUP, GROUP), jnp.int32),
            pltpu.VMEM((BUFROWS, EMB), jnp.float32),
            pltpu.SemaphoreType.DMA,
        ],
    )
    def gather_kernel(table_hbm, idx_hbm, out_hbm, idx_v, buf, gsem):
        wid = lax.axis_index("s") * 2 + lax.axis_index("c")
        pltpu.sync_copy(idx_hbm.at[wid], idx_v)

        @pl.loop(0, NOUT)
        def _outer(i):
            copies = [
                pltpu.async_copy(
                    table_hbm.at[idx_v.at[i * NBUF_G + t]],
                    buf.at[pl.ds(t * GROUP, GROUP)],
                    gsem,
                )
                for t in range(NBUF_G)
            ]
            for c in copies:
                c.wait()
            pltpu.sync_copy(
                buf, out_hbm.at[pl.ds(wid * RPW + i * BUFROWS, BUFROWS)]
            )

    return gather_kernel(table_flat, idx3)


def _stats_part(h):
    return jnp.concatenate(
        [
            jnp.sum(h, axis=0, keepdims=True),
            jnp.sum(h * h, axis=0, keepdims=True),
            jnp.zeros((6, h.shape[1]), jnp.float32),
        ],
        axis=0,
    )


def _l1_body(xc_ref, num_ref, w_ref, wl_ref, b_ref, h_ref, st_ref):
    i = pl.program_id(0)
    acc = jnp.dot(xc_ref[...], w_ref[...], preferred_element_type=jnp.float32)
    acc = acc + num_ref[...] * wl_ref[...] + b_ref[...]
    h = jnp.maximum(acc, 0.0)
    h_ref[...] = h

    @pl.when(i == 0)
    def _():
        st_ref[...] = jnp.zeros_like(st_ref)

    st_ref[...] += _stats_part(h)


def _mid_body(h_ref, st_ref, g_ref, be_ref, w_ref, b_ref, o_ref, st2_ref):
    i = pl.program_id(0)
    mean = st_ref[0:1, :] * (1.0 / BATCH)
    var = st_ref[1:2, :] * (1.0 / BATCH) - mean * mean
    inv = g_ref[...] * lax.rsqrt(var + EPS)
    xn = (h_ref[...] - mean) * inv + be_ref[...]
    acc = jnp.dot(xn, w_ref[...], preferred_element_type=jnp.float32) + b_ref[...]
    h = jnp.maximum(acc, 0.0)
    o_ref[...] = h

    @pl.when(i == 0)
    def _():
        st2_ref[...] = jnp.zeros_like(st2_ref)

    st2_ref[...] += _stats_part(h)


def _fin_body(h_ref, st_ref, g_ref, be_ref, wo_ref, wide_ref, w1_ref, b1_ref,
              w2_ref, bf_ref, o_ref):
    mean = st_ref[0:1, :] * (1.0 / BATCH)
    var = st_ref[1:2, :] * (1.0 / BATCH) - mean * mean
    inv = g_ref[...] * lax.rsqrt(var + EPS)
    xn = (h_ref[...] - mean) * inv + be_ref[...]
    deep = jnp.dot(xn, wo_ref[...], preferred_element_type=jnp.float32)
    wh = jnp.maximum(
        jnp.dot(wide_ref[...], w1_ref[...], preferred_element_type=jnp.float32)
        + b1_ref[...],
        0.0,
    )
    wo = jnp.dot(wh, w2_ref[...], preferred_element_type=jnp.float32)
    o_ref[...] = jax.nn.sigmoid(deep + wo + bf_ref[...])


def _full(shape):
    return pl.BlockSpec(shape, lambda i: (0, 0))


def kernel(wide_input, deep_numerical_inputs, cat_inputs, params):
    p = params
    cat = cat_inputs.astype(jnp.int32)
    offs = jnp.arange(NF, dtype=jnp.int32) * NV
    idx3 = (cat + offs[None, :]).reshape(NW, RPW // GROUP, GROUP)
    table_flat = p["emb_tables"].reshape(NF * NV, EMB)

    rows = _sc_gather(table_flat, idx3)
    xc = rows.reshape(BATCH, DW)

    w0 = p["W_0"]
    h1, st1 = pl.pallas_call(
        _l1_body,
        grid=(NBLK,),
        in_specs=[
            pl.BlockSpec((BB, DW), lambda i: (i, 0)),
            pl.BlockSpec((BB, 1), lambda i: (i, 0)),
            _full((DW, 1024)),
            _full((1, 1024)),
            _full((1, 1024)),
        ],
        out_specs=[
            pl.BlockSpec((BB, 1024), lambda i: (i, 0)),
            _full((8, 1024)),
        ],
        out_shape=[
            jax.ShapeDtypeStruct((BATCH, 1024), jnp.float32),
            jax.ShapeDtypeStruct((8, 1024), jnp.float32),
        ],
    )(xc, deep_numerical_inputs, w0[:DW], w0[DW:DW + 1],
      p["b_0"].reshape(1, 1024))

    def mid(h, st, li, n_in, n_out):
        return pl.pallas_call(
            _mid_body,
            grid=(NBLK,),
            in_specs=[
                pl.BlockSpec((BB, n_in), lambda i: (i, 0)),
                _full((8, n_in)),
                _full((1, n_in)),
                _full((1, n_in)),
                _full((n_in, n_out)),
                _full((1, n_out)),
            ],
            out_specs=[
                pl.BlockSpec((BB, n_out), lambda i: (i, 0)),
                _full((8, n_out)),
            ],
            out_shape=[
                jax.ShapeDtypeStruct((BATCH, n_out), jnp.float32),
                jax.ShapeDtypeStruct((8, n_out), jnp.float32),
            ],
        )(h, st, p["bn_g_%d" % li].reshape(1, n_in),
          p["bn_b_%d" % li].reshape(1, n_in), p["W_%d" % (li + 1)],
          p["b_%d" % (li + 1)].reshape(1, n_out))

    h2, st2 = mid(h1, st1, 0, 1024, 512)
    h3, st3 = mid(h2, st2, 1, 512, 256)

    fw0 = p["final_W"][0, 0]
    fw1 = p["final_W"][1, 0]
    wout = p["W_out"] * fw1
    w2 = p["wide_W2"] * fw0
    bfin = (p["b_out"] * fw1 + p["wide_b2"] * fw0 + p["final_b"]).reshape(1, 1)

    out = pl.pallas_call(
        _fin_body,
        grid=(NBLK,),
        in_specs=[
            pl.BlockSpec((BB, 256), lambda i: (i, 0)),
            _full((8, 256)),
            _full((1, 256)),
            _full((1, 256)),
            _full((256, 1)),
            pl.BlockSpec((BB, 128), lambda i: (i, 0)),
            _full((128, 32)),
            _full((1, 32)),
            _full((32, 1)),
            _full((1, 1)),
        ],
        out_specs=pl.BlockSpec((BB, 1), lambda i: (i, 0)),
        out_shape=jax.ShapeDtypeStruct((BATCH, 1), jnp.float32),
    )(h3, st3, p["bn_g_2"].reshape(1, 256), p["bn_b_2"].reshape(1, 256),
      wout, wide_input, p["wide_W1"], p["wide_b1"].reshape(1, 32), w2, bfin)

    return out
